# Initial kernel scaffold; baseline (speedup 1.0000x reference)
import jax
import jax.numpy as jnp
from jax.experimental import pallas as pl


def kernel(x1, x2, params):
    raise NotImplementedError("write your pallas kernel here")


# speedup vs baseline: 1.0497x; 1.0497x over previous
import jax
import jax.numpy as jnp
from jax.experimental import pallas as pl
from jax.experimental.pallas import tpu as pltpu

_C = 384
_H = 56
_W = 56
_K = 192
_HID = 64
_HW = _H * _W
_F32 = jnp.float32


def _dot(a, b, dims):
    return jax.lax.dot_general(a, b, (dims, ((), ())),
                               preferred_element_type=_F32)


def _iota2(shape, dim):
    return jax.lax.broadcasted_iota(jnp.int32, shape, dim)


def _dw_body(x_ref, w5_ref, b0_ref, w7_ref, bsp_ref,
             w1t_ref, b1_ref, w2t_ref, b2_ref,
             b1o_ref, b2o_ref):
    x = x_ref[0]

    xb = x.astype(jnp.bfloat16).astype(_F32)
    p5 = jnp.pad(xb, ((2, 2), (2, 2), (0, 0)))
    a1 = jnp.zeros((_H, _W, _C), _F32)
    for dy in range(5):
        for dx in range(5):
            a1 = a1 + p5[dy:dy + _H, dx:dx + _W, :] * w5_ref[dy, dx, :][None, None, :]
    a1 = a1 + b0_ref[0][None, None, :]

    p7 = jnp.pad(a1, ((9, 9), (9, 9), (0, 0)))
    a2 = jnp.zeros((_H, _W, _C), _F32)
    for ty in range(7):
        for tx in range(7):
            a2 = a2 + (p7[3 * ty:3 * ty + _H, 3 * tx:3 * tx + _W, :]
                       * w7_ref[ty, tx, :][None, None, :])
    a2 = a2 + bsp_ref[0][None, None, :]

    b1o_ref[0] = _dot(a1.reshape(_HW, _C), w1t_ref[...], ((1,), (0,))) + b1_ref[0][None, :]
    b2o_ref[0] = _dot(a2.reshape(_HW, _C), w2t_ref[...], ((1,), (0,))) + b2_ref[0][None, :]


def _apply_body(x_ref, b1_ref, b2_ref, g0_ref, g1_ref, wct_ref, bc_ref,
                y_ref):
    att = b1_ref[0] * g0_ref[0] + b2_ref[0] * g1_ref[0]
    attf = _dot(att, wct_ref[...], ((1,), (0,))) + bc_ref[0][None, :]
    y_ref[0] = (x_ref[0].reshape(_HW, _C) * attf).reshape(_H, _W, _C)


def _exchange_body(x1_ref, x2_ref, m1r_ref, m1c_ref, m2r_ref, m2c_ref,
                   fc1_w_ref, fc1_b_ref, fc2_w_ref, fc2_b_ref,
                   out1_ref, out2_ref):
    x1 = x1_ref[0]
    x2 = x2_ref[0]

    ii = _iota2((_C, _C), 0)
    jj = _iota2((_C, _C), 1)
    tri = (jj <= ii).astype(_F32)
    jk = jax.lax.broadcasted_iota(jnp.int32, (_C, _K), 1).astype(_F32)

    def sel_mask(mr, mc):
        beats = (mr > mc) | ((mr == mc) & (jj < ii))
        rank = jnp.sum(beats.astype(_F32), axis=1, keepdims=True)
        selb = rank < float(_K)
        sel = selb.astype(_F32)
        csum = _dot(tri, sel, ((1,), (0,)))
        pos = jnp.where(selb, csum - 1.0, -1.0)
        return sel, (pos == jk).astype(_F32)

    sel1, mask1 = sel_mask(m1r_ref[0], m1c_ref[0])
    sel2, mask2 = sel_mask(m2r_ref[0], m2c_ref[0])

    w1e1t = _dot(fc1_w_ref[...], mask1, ((1,), (1,)))
    w1e2t = _dot(fc1_w_ref[...], mask2, ((1,), (1,)))
    w2e1 = _dot(mask1, fc2_w_ref[...], ((1,), (0,)))
    w2e2 = _dot(mask2, fc2_w_ref[...], ((1,), (0,)))
    be1 = _dot(mask1, fc2_b_ref[...], ((1,), (0,)))
    be2 = _dot(mask2, fc2_b_ref[...], ((1,), (0,)))

    b1 = fc1_b_ref[...]
    h1 = jnp.maximum(_dot(w1e1t, x1, ((1,), (0,))) + b1, 0.0)
    h2 = jnp.maximum(_dot(w1e2t, x2, ((1,), (0,))) + b1, 0.0)
    y1 = _dot(w2e1, h2, ((1,), (0,))) + be1
    y2 = _dot(w2e2, h1, ((1,), (0,))) + be2
    out1_ref[0] = jnp.where(sel1 > 0.5, y1, x1)
    out2_ref[0] = jnp.where(sel2 > 0.5, y2, x2)


def _full(shape):
    return pl.BlockSpec(shape, lambda i: tuple(0 for _ in shape))


def kernel(x1, x2, params):
    p = params
    nb = x1.shape[0]
    bf = lambda v: v.astype(jnp.bfloat16).astype(_F32)
    row = lambda v: v.reshape(1, -1)

    xt = jnp.concatenate([x1.transpose(0, 2, 3, 1), x2.transpose(0, 2, 3, 1)],
                         axis=0)
    w5 = bf(p['conv0_w']).reshape(_C, 5, 5).transpose(1, 2, 0)
    w7 = bf(p['convsp_w']).reshape(_C, 7, 7).transpose(1, 2, 0)
    w1t = p['conv1_w'].reshape(_K, _C).T
    w2t = p['conv2_w'].reshape(_K, _C).T
    wct = p['conv_w'].reshape(_C, _K).T

    hwc_spec = pl.BlockSpec((1, _H, _W, _C), lambda i: (i, 0, 0, 0))
    bmap_spec = pl.BlockSpec((1, _HW, _K), lambda i: (i, 0, 0))
    b1m, b2m = pl.pallas_call(
        _dw_body,
        grid=(2 * nb,),
        in_specs=[hwc_spec,
                  _full((5, 5, _C)), _full((1, _C)),
                  _full((7, 7, _C)), _full((1, _C)),
                  _full((_C, _K)), _full((1, _K)),
                  _full((_C, _K)), _full((1, _K))],
        out_specs=[bmap_spec, bmap_spec],
        out_shape=[jax.ShapeDtypeStruct((2 * nb, _HW, _K), _F32),
                   jax.ShapeDtypeStruct((2 * nb, _HW, _K), _F32)],
    )(xt, w5, row(p['conv0_b']), w7, row(p['convsp_b']),
      w1t, row(p['conv1_b']), w2t, row(p['conv2_b']))

    b1n = b1m.transpose(0, 2, 1).reshape(2 * nb, _K, _H, _W)
    b2n = b2m.transpose(0, 2, 1).reshape(2 * nb, _K, _H, _W)
    attn_cat = jnp.concatenate([b1n, b2n], axis=1)
    agg = jnp.concatenate([jnp.mean(attn_cat, axis=1, keepdims=True),
                           jnp.max(attn_cat, axis=1, keepdims=True)], axis=1)
    sig = jax.nn.sigmoid(jax.lax.conv_general_dilated(
        agg, p['convsq_w'], (1, 1), [(3, 3), (3, 3)],
        dimension_numbers=('NCHW', 'OIHW', 'NCHW'))
        + p['convsq_b'][None, :, None, None])
    g0 = sig[:, 0].reshape(2 * nb, _HW, 1)
    g1 = sig[:, 1].reshape(2 * nb, _HW, 1)

    y = pl.pallas_call(
        _apply_body,
        grid=(2 * nb,),
        in_specs=[hwc_spec, bmap_spec, bmap_spec,
                  pl.BlockSpec((1, _HW, 1), lambda i: (i, 0, 0)),
                  pl.BlockSpec((1, _HW, 1), lambda i: (i, 0, 0)),
                  _full((_K, _C)), _full((1, _C))],
        out_specs=hwc_spec,
        out_shape=jax.ShapeDtypeStruct((2 * nb, _H, _W, _C), _F32),
    )(xt, b1m, b2m, g0, g1, wct, row(p['conv_b']))

    m1 = jax.nn.sigmoid(jnp.mean(y[:nb].transpose(0, 3, 1, 2), axis=(2, 3)))
    m2 = jax.nn.sigmoid(jnp.mean(y[nb:].transpose(0, 3, 1, 2), axis=(2, 3)))

    x1r = x1.reshape(nb, _C, _HW)
    x2r = x2.reshape(nb, _C, _HW)
    out1, out2 = pl.pallas_call(
        _exchange_body,
        grid=(nb,),
        in_specs=[
            pl.BlockSpec((1, _C, _HW), lambda n: (n, 0, 0)),
            pl.BlockSpec((1, _C, _HW), lambda n: (n, 0, 0)),
            pl.BlockSpec((1, 1, _C), lambda n: (n, 0, 0)),
            pl.BlockSpec((1, _C, 1), lambda n: (n, 0, 0)),
            pl.BlockSpec((1, 1, _C), lambda n: (n, 0, 0)),
            pl.BlockSpec((1, _C, 1), lambda n: (n, 0, 0)),
            _full((_HID, _K)), _full((_HID, 1)),
            _full((_K, _HID)), _full((_K, 1)),
        ],
        out_specs=[pl.BlockSpec((1, _C, _HW), lambda n: (n, 0, 0)),
                   pl.BlockSpec((1, _C, _HW), lambda n: (n, 0, 0))],
        out_shape=[jax.ShapeDtypeStruct((nb, _C, _HW), _F32),
                   jax.ShapeDtypeStruct((nb, _C, _HW), _F32)],
    )(x1r, x2r, m1[:, None, :], m1[:, :, None], m2[:, None, :], m2[:, :, None],
      p['fc1_w'], p['fc1_b'].reshape(_HID, 1),
      p['fc2_w'], p['fc2_b'].reshape(_K, 1))

    return (out1.reshape(nb, _C, _H, _W), out2.reshape(nb, _C, _H, _W))
